# Initial kernel scaffold; baseline (speedup 1.0000x reference)
import jax
import jax.numpy as jnp
from jax.experimental import pallas as pl


def kernel(x, edge_index, W1, b1, g1, be1, W2, b2, g2, be2, W3, b3):
    raise NotImplementedError("write your pallas kernel here")


# speedup vs baseline: 21.7525x; 21.7525x over previous
import functools
import math

import jax
import jax.numpy as jnp
from jax import lax
from jax.experimental import pallas as pl
from jax.experimental.pallas import tpu as pltpu
from jax.experimental.pallas import tpu_sc as plsc

N = 10000
NP = 10240
E = 320000
CH = 128
NWIN = E // CH
WBASE = 80
NLAST = NWIN - 31 * WBASE
PHW = 40
PH = WBASE // PHW
RPT = NP // 16
D = 128
BLK = 512
GRID = NP // BLK
BN_C = 1.0 / math.sqrt(1.0 + 1e-5)

_MESH = plsc.VectorSubcoreMesh(core_axis_name="c", subcore_axis_name="s")


def _stage_windows(edge_hbm, buf, w, p):
    @pl.when(w < 31)
    def _():
        pltpu.sync_copy(edge_hbm.at[pl.ds(w * WBASE + p * PHW, PHW)], buf)

    if p == 0:

        @pl.when(w == 31)
        def _():
            pltpu.sync_copy(edge_hbm.at[pl.ds(31 * WBASE, NLAST)],
                            buf.at[pl.ds(0, NLAST)])


def _zero_rows(zbuf, acc, base):
    width = zbuf.shape[1]

    @pl.loop(0, CH)
    def _(r):
        row = zbuf.at[r]
        for k in range(width // 16):
            row[pl.ds(k * 16, 16)] = jnp.zeros((16,), jnp.float32)

    for t in range(RPT // CH):
        pltpu.sync_copy(zbuf, acc.at[pl.ds(base + t * CH, CH)])


@functools.partial(
    pl.kernel,
    out_type=jax.ShapeDtypeStruct((2, NP, D), jnp.float32),
    mesh=_MESH,
    scratch_types=[
        pltpu.VMEM((PHW, CH), jnp.int32),
        pltpu.VMEM((PHW, CH), jnp.int32),
        pltpu.VMEM((2, CH, D), jnp.float32),
        pltpu.VMEM_SHARED((NP, D), jnp.float32),
        pltpu.SemaphoreType.DMA,
    ],
)
def _agg(edge_hbm, tab_hbm, out_hbm, src_v, dst_v, gbuf, acc, gsem):
    c = lax.axis_index("c")
    s = lax.axis_index("s")
    w = s * 2 + c
    nw = jnp.where(w < 31, WBASE, NLAST)
    base = s * RPT
    _zero_rows(gbuf.at[0], acc, base)
    plsc.subcore_barrier()

    for p in range(PH):
        _stage_windows(edge_hbm.at[0], src_v, w, p)
        _stage_windows(edge_hbm.at[1], dst_v, w, p)
        nph = jnp.clip(nw - p * PHW, 0, PHW)

        @pl.when(nph > 0)
        def _(nph=nph):
            pltpu.async_copy(tab_hbm.at[src_v.at[0]], gbuf.at[0], gsem)

            @pl.loop(0, nph)
            def _(j):
                b = j % 2
                pltpu.make_async_copy(tab_hbm.at[src_v.at[j]], gbuf.at[b],
                                      gsem).wait()

                @pl.when(j + 1 < nph)
                def _():
                    pltpu.async_copy(tab_hbm.at[src_v.at[j + 1]],
                                     gbuf.at[(j + 1) % 2], gsem)

                pltpu.sync_copy(gbuf.at[b], acc.at[dst_v.at[j]], add=True)

    plsc.subcore_barrier()
    pltpu.sync_copy(acc.at[pl.ds(base, RPT)],
                    out_hbm.at[c].at[pl.ds(base, RPT)])


@functools.partial(
    pl.kernel,
    out_type=jax.ShapeDtypeStruct((2, NP, D), jnp.float32),
    mesh=_MESH,
    scratch_types=[
        pltpu.VMEM((PHW, CH), jnp.int32),
        pltpu.VMEM((2, CH, D), jnp.float32),
        pltpu.VMEM_SHARED((NP, D), jnp.float32),
    ],
)
def _deg_kernel(edge_hbm, out_hbm, dst_v, obuf, acc):
    c = lax.axis_index("c")
    s = lax.axis_index("s")
    w = s * 2 + c
    nw = jnp.where(w < 31, WBASE, NLAST)
    base = s * RPT

    @pl.loop(0, CH)
    def _(r):
        for k in range(D // 16):
            obuf.at[0].at[r][pl.ds(k * 16, 16)] = jnp.zeros((16,), jnp.float32)
            obuf.at[1].at[r][pl.ds(k * 16, 16)] = jnp.ones((16,), jnp.float32)

    for t in range(RPT // CH):
        pltpu.sync_copy(obuf.at[0], acc.at[pl.ds(base + t * CH, CH)])
    plsc.subcore_barrier()

    ones = obuf.at[1]
    for p in range(PH):
        _stage_windows(edge_hbm.at[1], dst_v, w, p)
        nph = jnp.clip(nw - p * PHW, 0, PHW)

        @pl.loop(0, nph)
        def _(j):
            pltpu.sync_copy(ones, acc.at[dst_v.at[j]], add=True)

    plsc.subcore_barrier()
    pltpu.sync_copy(acc.at[pl.ds(base, RPT)],
                    out_hbm.at[c].at[pl.ds(base, RPT)])


def _dinv_block(deg_ref, i):
    deg = deg_ref[0, :, :] + deg_ref[1, :, :] + 1.0
    rows = i * BLK + lax.broadcasted_iota(jnp.int32, (BLK, D), 0)
    dinv = jnp.where(rows < N, lax.rsqrt(deg), 0.0)
    return dinv[:, 0:1]


def _tc1_body(deg_ref, x_ref, w_ref, p_ref):
    i = pl.program_id(0)
    dinv = _dinv_block(deg_ref, i)
    h = jnp.dot(x_ref[...], w_ref[...], preferred_element_type=jnp.float32)
    rows = i * BLK + lax.broadcasted_iota(jnp.int32, (BLK, 1), 0)
    p_ref[...] = jnp.where(rows < N, h * dinv, 0.0)


def _tc_mid_body(deg_ref, s_ref, p_ref, w_ref, b_ref, g_ref, be_ref, o_ref):
    dinv = _dinv_block(deg_ref, pl.program_id(0))
    conv = (s_ref[0] + s_ref[1] + p_ref[...]) * dinv + b_ref[...]
    a = jnp.maximum(conv * (g_ref[...] * BN_C) + be_ref[...], 0.0)
    o_ref[...] = jnp.dot(a, w_ref[...],
                         preferred_element_type=jnp.float32) * dinv


def _tc3_body(deg_ref, s_ref, p_ref, b_ref, g_ref, be_ref, o_ref):
    dinv = _dinv_block(deg_ref, pl.program_id(0))
    conv = (s_ref[0] + s_ref[1] + p_ref[...]) * dinv + b_ref[...]
    a = jnp.maximum(conv * (g_ref[...] * BN_C) + be_ref[...], 0.0)
    o_ref[...] = a * dinv


def _tc_out_body(deg_ref, s_ref, p_ref, w_ref, b_ref, o_ref):
    dinv = _dinv_block(deg_ref, pl.program_id(0))
    agg = (s_ref[0] + s_ref[1] + p_ref[...]) * dinv
    conv = jnp.dot(agg, w_ref[...],
                   preferred_element_type=jnp.float32) + b_ref[...]
    m = jnp.max(conv, axis=1, keepdims=True)
    lse = jnp.log(jnp.sum(jnp.exp(conv - m), axis=1, keepdims=True)) + m
    o_ref[...] = conv - lse


def _deg_spec():
    return pl.BlockSpec((2, BLK, D), lambda i: (0, i, 0))


def _s_spec():
    return pl.BlockSpec((2, BLK, D), lambda i: (0, i, 0))


def _p_spec():
    return pl.BlockSpec((BLK, D), lambda i: (i, 0))


def _full_spec(r, c):
    return pl.BlockSpec((r, c), lambda i: (0, 0))


def _tc1(degp, x, w1):
    return pl.pallas_call(
        _tc1_body,
        grid=(GRID,),
        in_specs=[_deg_spec(), _p_spec(), _full_spec(D, D)],
        out_specs=_p_spec(),
        out_shape=jax.ShapeDtypeStruct((NP, D), jnp.float32),
    )(degp, x, w1)


def _tc_mid(degp, s_in, p_in, w, b, g, be):
    return pl.pallas_call(
        _tc_mid_body,
        grid=(GRID,),
        in_specs=[_deg_spec(), _s_spec(), _p_spec(), _full_spec(D, D),
                  _full_spec(1, D), _full_spec(1, D), _full_spec(1, D)],
        out_specs=_p_spec(),
        out_shape=jax.ShapeDtypeStruct((NP, D), jnp.float32),
    )(degp, s_in, p_in, w, b, g, be)


def _tc3(degp, s_in, p_in, b, g, be):
    return pl.pallas_call(
        _tc3_body,
        grid=(GRID,),
        in_specs=[_deg_spec(), _s_spec(), _p_spec(),
                  _full_spec(1, D), _full_spec(1, D), _full_spec(1, D)],
        out_specs=_p_spec(),
        out_shape=jax.ShapeDtypeStruct((NP, D), jnp.float32),
    )(degp, s_in, p_in, b, g, be)


def _tc_out(degp, s_in, p_in, w3, b3):
    return pl.pallas_call(
        _tc_out_body,
        grid=(GRID,),
        in_specs=[_deg_spec(), _s_spec(), _p_spec(), _full_spec(D, 64),
                  _full_spec(1, 64)],
        out_specs=pl.BlockSpec((BLK, 64), lambda i: (i, 0)),
        out_shape=jax.ShapeDtypeStruct((N, 64), jnp.float32),
    )(degp, s_in, p_in, w3, b3)


def kernel(x, edge_index, W1, b1, g1, be1, W2, b2, g2, be2, W3, b3):
    edges = edge_index.astype(jnp.int32).reshape(2, NWIN, CH)
    b1r, g1r, be1r = b1.reshape(1, -1), g1.reshape(1, -1), be1.reshape(1, -1)
    b2r, g2r, be2r = b2.reshape(1, -1), g2.reshape(1, -1), be2.reshape(1, -1)
    b3r = b3.reshape(1, -1)

    degp = _deg_kernel(edges)
    p1 = _tc1(degp, x, W1)
    s1 = _agg(edges, p1)
    p2 = _tc_mid(degp, s1, p1, W2, b1r, g1r, be1r)
    s2 = _agg(edges, p2)
    p3 = _tc3(degp, s2, p2, b2r, g2r, be2r)
    s3 = _agg(edges, p3)
    out = _tc_out(degp, s3, p3, W3, b3r)
    return out


# speedup vs baseline: 24.8094x; 1.1405x over previous
import functools
import math

import jax
import jax.numpy as jnp
from jax import lax
from jax.experimental import pallas as pl
from jax.experimental.pallas import tpu as pltpu
from jax.experimental.pallas import tpu_sc as plsc

N = 10000
NP = 10240
E = 320000
CH = 128
NWIN = E // CH
WBASE = 80
NLAST = NWIN - 31 * WBASE
PHW = 40
PH = WBASE // PHW
RPT = NP // 16
D = 128
BLK = 512
GRID = NP // BLK
BN_C = 1.0 / math.sqrt(1.0 + 1e-5)

_MESH = plsc.VectorSubcoreMesh(core_axis_name="c", subcore_axis_name="s")


def _stage_windows(edge_hbm, buf, w, p):
    @pl.when(w < 31)
    def _():
        pltpu.sync_copy(edge_hbm.at[pl.ds(w * WBASE + p * PHW, PHW)], buf)

    if p == 0:

        @pl.when(w == 31)
        def _():
            pltpu.sync_copy(edge_hbm.at[pl.ds(31 * WBASE, NLAST)],
                            buf.at[pl.ds(0, NLAST)])


def _zero_rows(zbuf, acc, base):
    width = zbuf.shape[1]

    @pl.loop(0, CH)
    def _(r):
        row = zbuf.at[r]
        for k in range(width // 16):
            row[pl.ds(k * 16, 16)] = jnp.zeros((16,), jnp.float32)

    for t in range(RPT // CH):
        pltpu.sync_copy(zbuf, acc.at[pl.ds(base + t * CH, CH)])


@functools.partial(
    pl.kernel,
    out_type=jax.ShapeDtypeStruct((2, NP, D), jnp.float32),
    mesh=_MESH,
    scratch_types=[
        pltpu.VMEM((PHW, CH), jnp.int32),
        pltpu.VMEM((PHW, CH), jnp.int32),
        pltpu.VMEM((2, CH, D), jnp.float32),
        pltpu.VMEM_SHARED((NP, D), jnp.float32),
        pltpu.SemaphoreType.DMA,
    ],
)
def _agg(edge_hbm, tab_hbm, out_hbm, src_v, dst_v, gbuf, acc, gsem):
    c = lax.axis_index("c")
    s = lax.axis_index("s")
    w = s * 2 + c
    nw = jnp.where(w < 31, WBASE, NLAST)
    base = s * RPT
    _zero_rows(gbuf.at[0], acc, base)
    plsc.subcore_barrier()

    for p in range(PH):
        _stage_windows(edge_hbm.at[0], src_v, w, p)
        _stage_windows(edge_hbm.at[1], dst_v, w, p)
        nph = jnp.clip(nw - p * PHW, 0, PHW)

        @pl.when(nph > 0)
        def _(nph=nph):
            pltpu.async_copy(tab_hbm.at[src_v.at[0]], gbuf.at[0], gsem)

            @pl.when(nph > 1)
            def _():
                pltpu.async_copy(tab_hbm.at[src_v.at[1]], gbuf.at[1], gsem)

            @pl.loop(0, nph)
            def _(j):
                b = j % 2
                pltpu.make_async_copy(tab_hbm.at[src_v.at[j]], gbuf.at[b],
                                      gsem).wait()
                pltpu.sync_copy(gbuf.at[b], acc.at[dst_v.at[j]], add=True)

                @pl.when(j + 2 < nph)
                def _():
                    pltpu.async_copy(tab_hbm.at[src_v.at[j + 2]],
                                     gbuf.at[b], gsem)

    plsc.subcore_barrier()
    pltpu.sync_copy(acc.at[pl.ds(base, RPT)],
                    out_hbm.at[c].at[pl.ds(base, RPT)])


@functools.partial(
    pl.kernel,
    out_type=jax.ShapeDtypeStruct((2, NP, D), jnp.float32),
    mesh=_MESH,
    scratch_types=[
        pltpu.VMEM((PHW, CH), jnp.int32),
        pltpu.VMEM((2, CH, D), jnp.float32),
        pltpu.VMEM_SHARED((NP, D), jnp.float32),
    ],
)
def _deg_kernel(edge_hbm, out_hbm, dst_v, obuf, acc):
    c = lax.axis_index("c")
    s = lax.axis_index("s")
    w = s * 2 + c
    nw = jnp.where(w < 31, WBASE, NLAST)
    base = s * RPT

    @pl.loop(0, CH)
    def _(r):
        for k in range(D // 16):
            obuf.at[0].at[r][pl.ds(k * 16, 16)] = jnp.zeros((16,), jnp.float32)
            obuf.at[1].at[r][pl.ds(k * 16, 16)] = jnp.ones((16,), jnp.float32)

    for t in range(RPT // CH):
        pltpu.sync_copy(obuf.at[0], acc.at[pl.ds(base + t * CH, CH)])
    plsc.subcore_barrier()

    ones = obuf.at[1]
    for p in range(PH):
        _stage_windows(edge_hbm.at[1], dst_v, w, p)
        nph = jnp.clip(nw - p * PHW, 0, PHW)

        @pl.loop(0, nph)
        def _(j):
            pltpu.sync_copy(ones, acc.at[dst_v.at[j]], add=True)

    plsc.subcore_barrier()
    pltpu.sync_copy(acc.at[pl.ds(base, RPT)],
                    out_hbm.at[c].at[pl.ds(base, RPT)])


def _dinv_block(deg_ref, i):
    deg = deg_ref[0, :, :] + deg_ref[1, :, :] + 1.0
    rows = i * BLK + lax.broadcasted_iota(jnp.int32, (BLK, D), 0)
    dinv = jnp.where(rows < N, lax.rsqrt(deg), 0.0)
    return dinv[:, 0:1]


def _tc1_body(deg_ref, x_ref, w_ref, p_ref):
    i = pl.program_id(0)
    dinv = _dinv_block(deg_ref, i)
    h = jnp.dot(x_ref[...], w_ref[...], preferred_element_type=jnp.float32)
    rows = i * BLK + lax.broadcasted_iota(jnp.int32, (BLK, 1), 0)
    p_ref[...] = jnp.where(rows < N, h * dinv, 0.0)


def _tc_mid_body(deg_ref, s_ref, p_ref, w_ref, b_ref, g_ref, be_ref, o_ref):
    dinv = _dinv_block(deg_ref, pl.program_id(0))
    conv = (s_ref[0] + s_ref[1] + p_ref[...]) * dinv + b_ref[...]
    a = jnp.maximum(conv * (g_ref[...] * BN_C) + be_ref[...], 0.0)
    o_ref[...] = jnp.dot(a, w_ref[...],
                         preferred_element_type=jnp.float32) * dinv


def _tc3_body(deg_ref, s_ref, p_ref, b_ref, g_ref, be_ref, o_ref):
    dinv = _dinv_block(deg_ref, pl.program_id(0))
    conv = (s_ref[0] + s_ref[1] + p_ref[...]) * dinv + b_ref[...]
    a = jnp.maximum(conv * (g_ref[...] * BN_C) + be_ref[...], 0.0)
    o_ref[...] = a * dinv


def _tc_out_body(deg_ref, s_ref, p_ref, w_ref, b_ref, o_ref):
    dinv = _dinv_block(deg_ref, pl.program_id(0))
    agg = (s_ref[0] + s_ref[1] + p_ref[...]) * dinv
    conv = jnp.dot(agg, w_ref[...],
                   preferred_element_type=jnp.float32) + b_ref[...]
    m = jnp.max(conv, axis=1, keepdims=True)
    lse = jnp.log(jnp.sum(jnp.exp(conv - m), axis=1, keepdims=True)) + m
    o_ref[...] = conv - lse


def _deg_spec():
    return pl.BlockSpec((2, BLK, D), lambda i: (0, i, 0))


def _s_spec():
    return pl.BlockSpec((2, BLK, D), lambda i: (0, i, 0))


def _p_spec():
    return pl.BlockSpec((BLK, D), lambda i: (i, 0))


def _full_spec(r, c):
    return pl.BlockSpec((r, c), lambda i: (0, 0))


def _tc1(degp, x, w1):
    return pl.pallas_call(
        _tc1_body,
        grid=(GRID,),
        in_specs=[_deg_spec(), _p_spec(), _full_spec(D, D)],
        out_specs=_p_spec(),
        out_shape=jax.ShapeDtypeStruct((NP, D), jnp.float32),
    )(degp, x, w1)


def _tc_mid(degp, s_in, p_in, w, b, g, be):
    return pl.pallas_call(
        _tc_mid_body,
        grid=(GRID,),
        in_specs=[_deg_spec(), _s_spec(), _p_spec(), _full_spec(D, D),
                  _full_spec(1, D), _full_spec(1, D), _full_spec(1, D)],
        out_specs=_p_spec(),
        out_shape=jax.ShapeDtypeStruct((NP, D), jnp.float32),
    )(degp, s_in, p_in, w, b, g, be)


def _tc3(degp, s_in, p_in, b, g, be):
    return pl.pallas_call(
        _tc3_body,
        grid=(GRID,),
        in_specs=[_deg_spec(), _s_spec(), _p_spec(),
                  _full_spec(1, D), _full_spec(1, D), _full_spec(1, D)],
        out_specs=_p_spec(),
        out_shape=jax.ShapeDtypeStruct((NP, D), jnp.float32),
    )(degp, s_in, p_in, b, g, be)


def _tc_out(degp, s_in, p_in, w3, b3):
    return pl.pallas_call(
        _tc_out_body,
        grid=(GRID,),
        in_specs=[_deg_spec(), _s_spec(), _p_spec(), _full_spec(D, 64),
                  _full_spec(1, 64)],
        out_specs=pl.BlockSpec((BLK, 64), lambda i: (i, 0)),
        out_shape=jax.ShapeDtypeStruct((N, 64), jnp.float32),
    )(degp, s_in, p_in, w3, b3)


def kernel(x, edge_index, W1, b1, g1, be1, W2, b2, g2, be2, W3, b3):
    edges = edge_index.astype(jnp.int32).reshape(2, NWIN, CH)
    b1r, g1r, be1r = b1.reshape(1, -1), g1.reshape(1, -1), be1.reshape(1, -1)
    b2r, g2r, be2r = b2.reshape(1, -1), g2.reshape(1, -1), be2.reshape(1, -1)
    b3r = b3.reshape(1, -1)

    degp = _deg_kernel(edges)
    p1 = _tc1(degp, x, W1)
    s1 = _agg(edges, p1)
    p2 = _tc_mid(degp, s1, p1, W2, b1r, g1r, be1r)
    s2 = _agg(edges, p2)
    p3 = _tc3(degp, s2, p2, b2r, g2r, be2r)
    s3 = _agg(edges, p3)
    out = _tc_out(degp, s3, p3, W3, b3r)
    return out
